# baseline (speedup 1.0000x reference)
import jax
import jax.numpy as jnp
from jax.experimental import pallas as pl
from jax.experimental.pallas import tpu as pltpu

B, N, D = 32, 128, 128
G = 16
K = 2 * G
SIGMA = 2.5


def _body(c_o, ct_o, x_o, c_m, ct_m, x_m,
          w1, b1, w2, b2, wt1, bt1, wt2, bt2, out, scr_an, scr_z, scr_h):
    w1v = w1[...].astype(jnp.bfloat16)
    w2v = w2[...].astype(jnp.bfloat16)
    b1v = b1[...]
    b2v = b2[...]

    ii = jax.lax.broadcasted_iota(jnp.int32, (N, N), 0)
    jj = jax.lax.broadcasted_iota(jnp.int32, (N, N), 1)
    eyef = jnp.where(ii == jj, jnp.float32(1.0), jnp.float32(0.0))
    maskf = 1.0 - eyef

    def adj(c, ct):
        d2 = (c[:, 0:1] - ct[0:1, :]) ** 2
        d2 += (c[:, 1:2] - ct[1:2, :]) ** 2
        d2 += (c[:, 2:3] - ct[2:3, :]) ** 2
        dist = jnp.sqrt(d2 + 1e-12)
        A = jnp.exp(dist * jnp.float32(-1.0 / SIGMA))
        A = A * maskf + eyef
        dinv_r = 1.0 / jnp.sqrt(jnp.sum(A, axis=1, keepdims=True))
        dinv_c = 1.0 / jnp.sqrt(jnp.sum(A, axis=0, keepdims=True))
        return ((A * dinv_r) * dinv_c).astype(jnp.bfloat16)

    for g in range(G):
        an_o = adj(c_o[g], ct_o[g])
        scr_an[g] = an_o
        scr_z[g] = jnp.dot(an_o, x_o[g].astype(jnp.bfloat16),
                           preferred_element_type=jnp.float32)
        an_m = adj(c_m[g], ct_m[g])
        scr_an[G + g] = an_m
        scr_z[G + g] = jnp.dot(an_m, x_m[g].astype(jnp.bfloat16),
                               preferred_element_type=jnp.float32)
    zb = scr_z[...].reshape(K * N, D).astype(jnp.bfloat16)
    h1 = jnp.maximum(
        jnp.dot(zb, w1v, preferred_element_type=jnp.float32) + b1v, 0.0)
    scr_h[...] = h1.astype(jnp.bfloat16).reshape(K, N, D)
    for k in range(K):
        scr_z[k] = jnp.dot(scr_an[k], scr_h[k],
                           preferred_element_type=jnp.float32)
    zb2 = scr_z[...].reshape(K * N, D).astype(jnp.bfloat16)
    h2 = jnp.maximum(
        jnp.dot(zb2, w2v, preferred_element_type=jnp.float32) + b2v, 0.0)

    po = jnp.concatenate(
        [jnp.max(h2[g * N:(g + 1) * N], axis=0, keepdims=True)
         for g in range(G)], axis=0)
    pm = jnp.concatenate(
        [jnp.max(h2[(G + g) * N:(G + g + 1) * N], axis=0, keepdims=True)
         for g in range(G)], axis=0)

    t = jnp.dot(po.astype(jnp.bfloat16), wt1[0:D, :].astype(jnp.bfloat16),
                preferred_element_type=jnp.float32)
    t += jnp.dot(pm.astype(jnp.bfloat16), wt1[D:2 * D, :].astype(jnp.bfloat16),
                 preferred_element_type=jnp.float32)
    t = jnp.maximum(t + bt1[...], 0.0)
    tb = t.astype(jnp.bfloat16).astype(jnp.float32)
    wb = wt2[...].astype(jnp.bfloat16).astype(jnp.float32)
    out[...] = jnp.sum(tb * wb, axis=1, keepdims=True) + bt2[0, 0]


def kernel(coords_orig, feats_orig, coords_mut, feats_mut,
           W1, b1, W2, b2, Wt1, bt1, Wt2, bt2):
    ct_o = jnp.swapaxes(coords_orig, 1, 2)
    ct_m = jnp.swapaxes(coords_mut, 1, 2)

    per_chunk = lambda i: (i, 0, 0)
    const2 = lambda i: (0, 0)

    return pl.pallas_call(
        _body,
        grid=(B // G,),
        in_specs=[
            pl.BlockSpec((G, N, 3), per_chunk),
            pl.BlockSpec((G, 3, N), per_chunk),
            pl.BlockSpec((G, N, D), per_chunk),
            pl.BlockSpec((G, N, 3), per_chunk),
            pl.BlockSpec((G, 3, N), per_chunk),
            pl.BlockSpec((G, N, D), per_chunk),
            pl.BlockSpec((D, D), const2),
            pl.BlockSpec((1, D), const2),
            pl.BlockSpec((D, D), const2),
            pl.BlockSpec((1, D), const2),
            pl.BlockSpec((2 * D, D), const2),
            pl.BlockSpec((1, D), const2),
            pl.BlockSpec((1, D), const2),
            pl.BlockSpec((1, 1), const2),
        ],
        out_specs=pl.BlockSpec((G, 1), lambda i: (i, 0)),
        out_shape=jax.ShapeDtypeStruct((B, 1), jnp.float32),
        scratch_shapes=[pltpu.VMEM((K, N, N), jnp.bfloat16),
                        pltpu.VMEM((K, N, D), jnp.float32),
                        pltpu.VMEM((K, N, D), jnp.bfloat16)],
        compiler_params=pltpu.CompilerParams(
            dimension_semantics=("arbitrary",)),
    )(coords_orig, ct_o, feats_orig, coords_mut, ct_m, feats_mut,
      W1, b1.reshape(1, D), W2, b2.reshape(1, D),
      Wt1, bt1.reshape(1, D), Wt2.reshape(1, D), bt2.reshape(1, 1))


# speedup vs baseline: 1.0515x; 1.0515x over previous
import jax
import jax.numpy as jnp
from jax.experimental import pallas as pl
from jax.experimental.pallas import tpu as pltpu

B, N, D = 32, 128, 128
G = 8
K = 2 * G
SIGMA = 2.5


def _body(c_o, ct_o, x_o, c_m, ct_m, x_m,
          w1, b1, w2, b2, wt1, bt1, wt2, bt2, out, scr_an, scr_z, scr_h):
    w1v = w1[...].astype(jnp.bfloat16)
    w2v = w2[...].astype(jnp.bfloat16)
    b1v = b1[...]
    b2v = b2[...]

    ii = jax.lax.broadcasted_iota(jnp.int32, (N, N), 0)
    jj = jax.lax.broadcasted_iota(jnp.int32, (N, N), 1)
    eyef = jnp.where(ii == jj, jnp.float32(1.0), jnp.float32(0.0))
    maskf = 1.0 - eyef

    def adj(c, ct):
        d2 = (c[:, 0:1] - ct[0:1, :]) ** 2
        d2 += (c[:, 1:2] - ct[1:2, :]) ** 2
        d2 += (c[:, 2:3] - ct[2:3, :]) ** 2
        dist = jnp.sqrt(d2 + 1e-12)
        A = jnp.exp(dist * jnp.float32(-1.0 / SIGMA))
        A = A * maskf + eyef
        dinv_r = 1.0 / jnp.sqrt(jnp.sum(A, axis=1, keepdims=True))
        dinv_c = 1.0 / jnp.sqrt(jnp.sum(A, axis=0, keepdims=True))
        return ((A * dinv_r) * dinv_c).astype(jnp.bfloat16)

    for g in range(G):
        an_o = adj(c_o[g], ct_o[g])
        scr_an[g] = an_o
        scr_z[g] = jnp.dot(an_o, x_o[g].astype(jnp.bfloat16),
                           preferred_element_type=jnp.float32)
        an_m = adj(c_m[g], ct_m[g])
        scr_an[G + g] = an_m
        scr_z[G + g] = jnp.dot(an_m, x_m[g].astype(jnp.bfloat16),
                               preferred_element_type=jnp.float32)
    H = K // 2
    for f in range(2):
        zb = scr_z[f * H:(f + 1) * H].reshape(H * N, D).astype(jnp.bfloat16)
        h1 = jnp.maximum(
            jnp.dot(zb, w1v, preferred_element_type=jnp.float32) + b1v, 0.0)
        scr_h[f * H:(f + 1) * H] = h1.astype(jnp.bfloat16).reshape(H, N, D)
        for k in range(f * H, (f + 1) * H):
            scr_z[k] = jnp.dot(scr_an[k], scr_h[k],
                               preferred_element_type=jnp.float32)
    pooled = [None] * K
    for f in range(2):
        zb2 = scr_z[f * H:(f + 1) * H].reshape(H * N, D).astype(jnp.bfloat16)
        h2 = jnp.maximum(
            jnp.dot(zb2, w2v, preferred_element_type=jnp.float32) + b2v, 0.0)
        for j in range(H):
            pooled[f * H + j] = jnp.max(h2[j * N:(j + 1) * N],
                                        axis=0, keepdims=True)
    po = jnp.concatenate(pooled[:G], axis=0)
    pm = jnp.concatenate(pooled[G:], axis=0)

    t = jnp.dot(po.astype(jnp.bfloat16), wt1[0:D, :].astype(jnp.bfloat16),
                preferred_element_type=jnp.float32)
    t += jnp.dot(pm.astype(jnp.bfloat16), wt1[D:2 * D, :].astype(jnp.bfloat16),
                 preferred_element_type=jnp.float32)
    t = jnp.maximum(t + bt1[...], 0.0)
    tb = t.astype(jnp.bfloat16).astype(jnp.float32)
    wb = wt2[...].astype(jnp.bfloat16).astype(jnp.float32)
    out[...] = jnp.sum(tb * wb, axis=1, keepdims=True) + bt2[0, 0]


def kernel(coords_orig, feats_orig, coords_mut, feats_mut,
           W1, b1, W2, b2, Wt1, bt1, Wt2, bt2):
    ct_o = jnp.swapaxes(coords_orig, 1, 2)
    ct_m = jnp.swapaxes(coords_mut, 1, 2)

    per_chunk = lambda i: (i, 0, 0)
    const2 = lambda i: (0, 0)

    return pl.pallas_call(
        _body,
        grid=(B // G,),
        in_specs=[
            pl.BlockSpec((G, N, 3), per_chunk),
            pl.BlockSpec((G, 3, N), per_chunk),
            pl.BlockSpec((G, N, D), per_chunk),
            pl.BlockSpec((G, N, 3), per_chunk),
            pl.BlockSpec((G, 3, N), per_chunk),
            pl.BlockSpec((G, N, D), per_chunk),
            pl.BlockSpec((D, D), const2),
            pl.BlockSpec((1, D), const2),
            pl.BlockSpec((D, D), const2),
            pl.BlockSpec((1, D), const2),
            pl.BlockSpec((2 * D, D), const2),
            pl.BlockSpec((1, D), const2),
            pl.BlockSpec((1, D), const2),
            pl.BlockSpec((1, 1), const2),
        ],
        out_specs=pl.BlockSpec((G, 1), lambda i: (i, 0)),
        out_shape=jax.ShapeDtypeStruct((B, 1), jnp.float32),
        scratch_shapes=[pltpu.VMEM((K, N, N), jnp.bfloat16),
                        pltpu.VMEM((K, N, D), jnp.float32),
                        pltpu.VMEM((K, N, D), jnp.bfloat16)],
        compiler_params=pltpu.CompilerParams(
            dimension_semantics=("arbitrary",)),
    )(coords_orig, ct_o, feats_orig, coords_mut, ct_m, feats_mut,
      W1, b1.reshape(1, D), W2, b2.reshape(1, D),
      Wt1, bt1.reshape(1, D), Wt2.reshape(1, D), bt2.reshape(1, 1))
